# baseline (speedup 1.0000x reference)
import math

import jax
import jax.numpy as jnp
from jax import lax
from jax.experimental import pallas as pl
from jax.experimental.pallas import tpu as pltpu
from jax.experimental.pallas import tpu_sc as plsc

VOCAB = 1000000
D_MODEL = 64
COEFF = math.sqrt(D_MODEL)

NC = 2
NS = 16
LANES = 16
NW = NC * NS

CHUNK = 128
NBUF = 2
PAIR = 2 * D_MODEL

_DNUMS = lax.GatherDimensionNumbers(
    offset_dims=(), collapsed_slice_dims=(0,), start_index_map=(0,)
)


def _splat_lane(v16, lane):
    idx = jnp.full((LANES, 1), 0, jnp.int32) + lane
    return lax.gather(
        v16, idx, _DNUMS, (1,), mode=lax.GatherScatterMode.PROMISE_IN_BOUNDS
    )


def _sc_gather(B):
    assert B % (NW * CHUNK) == 0
    b_per_w = B // NW
    G = b_per_w // CHUNK
    assert G % NBUF == 0 and G >= 2 * NBUF

    mesh = plsc.VectorSubcoreMesh(
        core_axis_name="c", subcore_axis_name="s", num_cores=NC, num_subcores=NS
    )

    def body(table_hbm, q_hbm, idx_hbm, out_hbm, q_v, idx_v, rows_in, rows_out,
             sem_g, sem_p):
        wid = lax.axis_index("s") * NC + lax.axis_index("c")
        prow0 = wid * (b_per_w // 2)

        pltpu.sync_copy(q_hbm.at[wid], q_v)
        pltpu.sync_copy(idx_hbm.at[wid], idx_v)

        def gather_start(g, b):
            pltpu.make_async_copy(
                table_hbm.at[q_v.at[g]], rows_in[b], sem_g[b]
            ).start()

        def gather_wait(g, b):
            pltpu.make_async_copy(
                table_hbm.at[q_v.at[g]], rows_in[b], sem_g[b]
            ).wait()

        def put_start(g, b):
            pltpu.make_async_copy(
                rows_out[b],
                out_hbm.at[pl.ds(prow0 + g * (CHUNK // 2), CHUNK // 2)],
                sem_p[b],
            ).start()

        def put_wait(g, b):
            pltpu.make_async_copy(
                rows_out[b],
                out_hbm.at[pl.ds(prow0 + g * (CHUNK // 2), CHUNK // 2)],
                sem_p[b],
            ).wait()

        def assemble_scale(g, b):
            src = rows_in[b]
            dst = rows_out[b]

            @plsc.parallel_loop(0, CHUNK, unroll=2)
            def _(p):
                idx16 = idx_v[g, pl.ds((p // LANES) * LANES, LANES)]
                par = (_splat_lane(idx16, p % LANES) % 2).astype(jnp.float32)
                o = (p % 2) * D_MODEL
                for c in range(D_MODEL // LANES):
                    lo = src[p, pl.ds(c * LANES, LANES)]
                    hi = src[p, pl.ds(D_MODEL + c * LANES, LANES)]
                    v = (lo + par * (hi - lo)) * COEFF
                    dst[p // 2, pl.ds(o + c * LANES, LANES)] = v

        def slot(i, t, first, last):
            b = t % NBUF
            gather_wait(i, b)
            if not first:
                put_wait(i - NBUF, b)
            assemble_scale(i, b)
            put_start(i, b)
            if not last:
                gather_start(i + NBUF, b)

        for t in range(NBUF):
            gather_start(t, t)
        for t in range(NBUF):
            slot(t, t, True, False)

        def loop_body(j, carry):
            i0 = j * NBUF
            for t in range(NBUF):
                slot(i0 + t, t, False, False)
            return carry

        lax.fori_loop(1, G // NBUF - 1, loop_body, 0)

        i0 = G - NBUF
        for t in range(NBUF):
            slot(i0 + t, t, False, True)
        for t in range(NBUF):
            put_wait(G - NBUF + t, t)

    kern = pl.kernel(
        body,
        out_type=jax.ShapeDtypeStruct((B // 2, PAIR), jnp.float32),
        mesh=mesh,
        compiler_params=pltpu.CompilerParams(use_tc_tiling_on_sc=False),
        scratch_types=[
            pltpu.VMEM((G, CHUNK), jnp.int32),
            pltpu.VMEM((G, CHUNK), jnp.int32),
            [pltpu.VMEM((CHUNK, PAIR), jnp.float32)] * NBUF,
            [pltpu.VMEM((CHUNK // 2, PAIR), jnp.float32)] * NBUF,
            [pltpu.SemaphoreType.DMA] * NBUF,
            [pltpu.SemaphoreType.DMA] * NBUF,
        ],
    )
    return kern, b_per_w


def kernel(x, table):
    I, J = x.shape
    B = x.size
    table_pairs = table.reshape(VOCAB // 2, PAIR)
    x_t = x.T.astype(jnp.int32)
    kern, b_per_w = _sc_gather(B)
    idx = x_t.reshape(NW, b_per_w // CHUNK, CHUNK)
    q = idx // 2
    g2 = kern(table_pairs, q, idx)
    g2 = g2.reshape(J, I, D_MODEL)
    return jnp.transpose(g2, (1, 0, 2))


# speedup vs baseline: 1.0020x; 1.0020x over previous
import math

import jax
import jax.numpy as jnp
from jax import lax
from jax.experimental import pallas as pl
from jax.experimental.pallas import tpu as pltpu
from jax.experimental.pallas import tpu_sc as plsc

VOCAB = 1000000
D_MODEL = 64
COEFF = math.sqrt(D_MODEL)

NC = 2
NS = 16
LANES = 16
NW = NC * NS

CHUNK = 128
NBUF = 2
PAIR = 2 * D_MODEL

_DNUMS = lax.GatherDimensionNumbers(
    offset_dims=(), collapsed_slice_dims=(0,), start_index_map=(0,)
)


def _splat_lane(v16, lane):
    idx = jnp.full((LANES, 1), 0, jnp.int32) + lane
    return lax.gather(
        v16, idx, _DNUMS, (1,), mode=lax.GatherScatterMode.PROMISE_IN_BOUNDS
    )


def _sc_gather(B):
    assert B % (NW * CHUNK) == 0
    b_per_w = B // NW
    G = b_per_w // CHUNK
    assert G % NBUF == 0 and G >= 2 * NBUF

    mesh = plsc.VectorSubcoreMesh(
        core_axis_name="c", subcore_axis_name="s", num_cores=NC, num_subcores=NS
    )

    def body(table_hbm, q_hbm, idx_hbm, out_hbm, q_v, idx_v, rows_in, rows_out,
             sem_g, sem_p):
        wid = lax.axis_index("s") * NC + lax.axis_index("c")
        prow0 = wid * (b_per_w // 2)

        pltpu.sync_copy(q_hbm.at[wid], q_v)
        pltpu.sync_copy(idx_hbm.at[wid], idx_v)

        def gather_start(g, b):
            pltpu.make_async_copy(
                table_hbm.at[q_v.at[g]], rows_in[b], sem_g[b]
            ).start()

        def gather_wait(g, b):
            pltpu.make_async_copy(
                table_hbm.at[q_v.at[g]], rows_in[b], sem_g[b]
            ).wait()

        def put_start(g, b):
            pltpu.make_async_copy(
                rows_out[b],
                out_hbm.at[pl.ds(prow0 + g * (CHUNK // 2), CHUNK // 2)],
                sem_p[b],
            ).start()

        def put_wait(g, b):
            pltpu.make_async_copy(
                rows_out[b],
                out_hbm.at[pl.ds(prow0 + g * (CHUNK // 2), CHUNK // 2)],
                sem_p[b],
            ).wait()

        def assemble_scale(g, b):
            src = rows_in[b]
            dst = rows_out[b]

            @plsc.parallel_loop(0, CHUNK, unroll=2)
            def _(p):
                idx16 = idx_v[g, pl.ds((p // LANES) * LANES, LANES)]
                par = (_splat_lane(idx16, p % LANES) % 2).astype(jnp.float32)
                o = (p % 2) * D_MODEL
                for c in range(D_MODEL // LANES):
                    lo = src[p, pl.ds(c * LANES, LANES)]
                    hi = src[p, pl.ds(D_MODEL + c * LANES, LANES)]
                    v = (lo + par * (hi - lo)) * COEFF
                    dst[p // 2, pl.ds(o + c * LANES, LANES)] = v

        def slot(i, t, first, last):
            b = t % NBUF
            gather_wait(i, b)
            if not first:
                put_wait(i - NBUF, b)
            assemble_scale(i, b)
            put_start(i, b)
            if not last:
                gather_start(i + NBUF, b)

        for t in range(NBUF):
            gather_start(t, t)
        for t in range(NBUF):
            slot(t, t, True, False)

        def loop_body(j, carry):
            i0 = j * NBUF
            for t in range(NBUF):
                slot(i0 + t, t, False, False)
            return carry

        lax.fori_loop(1, G // NBUF - 1, loop_body, 0)

        i0 = G - NBUF
        for t in range(NBUF):
            slot(i0 + t, t, False, True)
        for t in range(NBUF):
            put_wait(G - NBUF + t, t)

    kern = pl.kernel(
        body,
        out_type=jax.ShapeDtypeStruct((B // 2, PAIR), jnp.float32),
        mesh=mesh,
        compiler_params=pltpu.CompilerParams(use_tc_tiling_on_sc=True),
        scratch_types=[
            pltpu.VMEM((G, CHUNK), jnp.int32),
            pltpu.VMEM((G, CHUNK), jnp.int32),
            [pltpu.VMEM((CHUNK, PAIR), jnp.float32)] * NBUF,
            [pltpu.VMEM((CHUNK // 2, PAIR), jnp.float32)] * NBUF,
            [pltpu.SemaphoreType.DMA] * NBUF,
            [pltpu.SemaphoreType.DMA] * NBUF,
        ],
    )
    return kern, b_per_w


def kernel(x, table):
    I, J = x.shape
    B = x.size
    table_pairs = table.reshape(VOCAB // 2, PAIR)
    x_t = x.T.astype(jnp.int32)
    kern, b_per_w = _sc_gather(B)
    idx = x_t.reshape(NW, b_per_w // CHUNK, CHUNK)
    q = idx // 2
    g2 = kern(table_pairs, q, idx)
    g2 = g2.reshape(J, I, D_MODEL)
    return jnp.transpose(g2, (1, 0, 2))


# speedup vs baseline: 1.0901x; 1.0879x over previous
import math

import jax
import jax.numpy as jnp
from jax import lax
from jax.experimental import pallas as pl
from jax.experimental.pallas import tpu as pltpu
from jax.experimental.pallas import tpu_sc as plsc

VOCAB = 1000000
D_MODEL = 64
COEFF = math.sqrt(D_MODEL)

NC = 2
NS = 16
LANES = 16
NW = NC * NS

CHUNK = 128
NBUF = 4


def _sc_gather(B):
    assert B % (NW * CHUNK) == 0
    b_per_w = B // NW
    G = b_per_w // CHUNK
    assert G % NBUF == 0 and G >= 2 * NBUF

    mesh = plsc.VectorSubcoreMesh(
        core_axis_name="c", subcore_axis_name="s", num_cores=NC, num_subcores=NS
    )

    def body(table_hbm, idx_hbm, out_hbm, idx_v, rows_in, rows_out, sem_g, sem_p):
        wid = lax.axis_index("s") * NC + lax.axis_index("c")
        row0 = wid * b_per_w

        pltpu.sync_copy(idx_hbm.at[wid], idx_v)

        def gather_start(g, b):
            pltpu.make_async_copy(
                table_hbm.at[idx_v.at[g]], rows_in[b], sem_g[b]
            ).start()

        def gather_wait(g, b):
            pltpu.make_async_copy(
                table_hbm.at[idx_v.at[g]], rows_in[b], sem_g[b]
            ).wait()

        def put_start(g, b):
            pltpu.make_async_copy(
                rows_out[b], out_hbm.at[pl.ds(row0 + g * CHUNK, CHUNK)], sem_p[b]
            ).start()

        def put_wait(g, b):
            pltpu.make_async_copy(
                rows_out[b], out_hbm.at[pl.ds(row0 + g * CHUNK, CHUNK)], sem_p[b]
            ).wait()

        def scale(b):
            src = rows_in[b]
            dst = rows_out[b]

            @plsc.parallel_loop(0, CHUNK, unroll=8)
            def _(r):
                for c in range(D_MODEL // LANES):
                    sl = pl.ds(c * LANES, LANES)
                    dst[r, sl] = src[r, sl] * COEFF

        def slot(i, t, first, last):
            b = t % NBUF
            gather_wait(i, b)
            if not first:
                put_wait(i - NBUF, b)
            scale(b)
            put_start(i, b)
            if not last:
                gather_start(i + NBUF, b)

        for t in range(NBUF):
            gather_start(t, t)
        for t in range(NBUF):
            slot(t, t, True, False)

        def loop_body(j, carry):
            i0 = j * NBUF
            for t in range(NBUF):
                slot(i0 + t, t, False, False)
            return carry

        lax.fori_loop(1, G // NBUF - 1, loop_body, 0)

        i0 = G - NBUF
        for t in range(NBUF):
            slot(i0 + t, t, False, True)
        for t in range(NBUF):
            put_wait(G - NBUF + t, t)

    kern = pl.kernel(
        body,
        out_type=jax.ShapeDtypeStruct((B, D_MODEL), jnp.float32),
        mesh=mesh,
        compiler_params=pltpu.CompilerParams(use_tc_tiling_on_sc=False),
        scratch_types=[
            pltpu.VMEM((G, CHUNK), jnp.int32),
            [pltpu.VMEM((CHUNK, D_MODEL), jnp.float32)] * NBUF,
            [pltpu.VMEM((CHUNK, D_MODEL), jnp.float32)] * NBUF,
            [pltpu.SemaphoreType.DMA] * NBUF,
            [pltpu.SemaphoreType.DMA] * NBUF,
        ],
    )
    return kern, b_per_w


def kernel(x, table):
    I, J = x.shape
    B = x.size
    x_t = x.T.astype(jnp.int32)
    kern, b_per_w = _sc_gather(B)
    idx = x_t.reshape(NW, b_per_w // CHUNK, CHUNK)
    g2 = kern(table, idx).reshape(J, I, D_MODEL)
    return jnp.transpose(g2, (1, 0, 2))
